# Initial kernel scaffold; baseline (speedup 1.0000x reference)
import functools

import jax
import jax.numpy as jnp
from jax import lax
from jax.experimental import pallas as pl
from jax.experimental.pallas import tpu as pltpu
from jax.experimental.pallas import tpu_sc as plsc

N = 10000
E = 160000
D = 128
NC = 2
NS = 16
NW = NC * NS
C = 128
NCHUNK = 40
EW = C * NCHUNK
EPAD = EW * NW
PAD_DST = N
ACC = 10240
RPT = ACC // NS
CW = 16

_mesh = plsc.VectorSubcoreMesh(
    core_axis_name="c", subcore_axis_name="s", num_cores=NC, num_subcores=NS)


def _conv_body(with_cnt, *refs):
    if with_cnt:
        (x_hbm, src_hbm, dst_hbm, out_hbm, cnt_hbm,
         idx_s, idx_d, rows, ones_b, zb16, acc, cacc, sem) = refs
    else:
        (x_hbm, src_hbm, dst_hbm, out_hbm,
         idx_s, idx_d, rows, acc, sem) = refs

    c = lax.axis_index("c")
    s = lax.axis_index("s")
    w = s * NC + c

    zeros16 = jnp.zeros((16,), jnp.float32)
    ones16 = jnp.ones((16,), jnp.float32)

    def _zrows(i, _):
        rows[i // 8, pl.ds((i % 8) * 16, 16)] = zeros16
        return 0
    lax.fori_loop(0, C * 8, _zrows, 0)

    def _zacc(i, _):
        pltpu.sync_copy(rows, acc.at[pl.ds(s * RPT + i * C, C)])
        return 0
    lax.fori_loop(0, RPT // C, _zacc, 0)

    if with_cnt:
        def _fill16(i, _):
            ones_b[i, :] = ones16
            zb16[i, :] = zeros16
            return 0
        lax.fori_loop(0, C, _fill16, 0)

        def _zcacc(i, _):
            pltpu.sync_copy(zb16, cacc.at[pl.ds(s * RPT + i * C, C)])
            return 0
        lax.fori_loop(0, RPT // C, _zcacc, 0)

    plsc.subcore_barrier()

    pltpu.sync_copy(src_hbm.at[w], idx_s)
    pltpu.sync_copy(dst_hbm.at[w], idx_d)

    def _step(j, _):
        pltpu.async_copy(x_hbm.at[idx_s.at[j]], rows, sem).wait()
        pltpu.sync_copy(rows, acc.at[idx_d.at[j]], add=True)
        if with_cnt:
            pltpu.sync_copy(ones_b, cacc.at[idx_d.at[j]], add=True)
        return 0
    lax.fori_loop(0, NCHUNK, _step, 0)

    plsc.subcore_barrier()

    base = s * RPT

    def _rd(i, _):
        pltpu.sync_copy(acc.at[pl.ds(base + i * C, C)], rows)
        pltpu.sync_copy(rows, out_hbm.at[c, pl.ds(base + i * C, C)])
        return 0
    lax.fori_loop(0, RPT // C, _rd, 0)

    if with_cnt:
        def _rdc(i, _):
            pltpu.sync_copy(cacc.at[pl.ds(base + i * C, C)], zb16)
            pltpu.sync_copy(zb16, cnt_hbm.at[c, pl.ds(base + i * C, C)])
            return 0
        lax.fori_loop(0, RPT // C, _rdc, 0)


_conv_cnt = pl.kernel(
    functools.partial(_conv_body, True),
    out_type=[
        jax.ShapeDtypeStruct((NC, ACC, D), jnp.float32),
        jax.ShapeDtypeStruct((NC, ACC, CW), jnp.float32),
    ],
    mesh=_mesh,
    scratch_types=[
        pltpu.VMEM((NCHUNK, C), jnp.int32),
        pltpu.VMEM((NCHUNK, C), jnp.int32),
        pltpu.VMEM((C, D), jnp.float32),
        pltpu.VMEM((C, CW), jnp.float32),
        pltpu.VMEM((C, CW), jnp.float32),
        pltpu.VMEM_SHARED((ACC, D), jnp.float32),
        pltpu.VMEM_SHARED((ACC, CW), jnp.float32),
        pltpu.SemaphoreType.DMA,
    ],
)

_conv_nocnt = pl.kernel(
    functools.partial(_conv_body, False),
    out_type=[
        jax.ShapeDtypeStruct((NC, ACC, D), jnp.float32),
    ],
    mesh=_mesh,
    scratch_types=[
        pltpu.VMEM((NCHUNK, C), jnp.int32),
        pltpu.VMEM((NCHUNK, C), jnp.int32),
        pltpu.VMEM((C, D), jnp.float32),
        pltpu.VMEM_SHARED((ACC, D), jnp.float32),
        pltpu.SemaphoreType.DMA,
    ],
)


def _pad_edges(ei):
    pad = EPAD - E
    src = jnp.concatenate([ei[0], jnp.zeros((pad,), jnp.int32)])
    dst = jnp.concatenate([ei[1], jnp.full((pad,), PAD_DST, jnp.int32)])
    return src.reshape(NW, NCHUNK, C), dst.reshape(NW, NCHUNK, C)


_RB = 500


def _lin_body(a0, a1, c0, c1, x, wl, wr, b, o):
    cnt = jnp.maximum(c0[0, :, 0:1] + c1[0, :, 0:1], 1.0)
    mean = (a0[0] + a1[0]) / cnt
    h = (jnp.dot(mean, wl[...], preferred_element_type=jnp.float32)
         + jnp.dot(x[...], wr[...], preferred_element_type=jnp.float32)
         + b[...])
    o[...] = jnp.maximum(h, 0.0)


def _sage_linear(agg, cnt, x, wl, wr, b):
    return pl.pallas_call(
        _lin_body,
        grid=(N // _RB,),
        in_specs=[
            pl.BlockSpec((1, _RB, D), lambda i: (0, i, 0)),
            pl.BlockSpec((1, _RB, D), lambda i: (1, i, 0)),
            pl.BlockSpec((1, _RB, CW), lambda i: (0, i, 0)),
            pl.BlockSpec((1, _RB, CW), lambda i: (1, i, 0)),
            pl.BlockSpec((_RB, D), lambda i: (i, 0)),
            pl.BlockSpec((D, D), lambda i: (0, 0)),
            pl.BlockSpec((D, D), lambda i: (0, 0)),
            pl.BlockSpec((1, D), lambda i: (0, 0)),
        ],
        out_specs=pl.BlockSpec((_RB, D), lambda i: (i, 0)),
        out_shape=jax.ShapeDtypeStruct((N, D), jnp.float32),
    )(agg, agg, cnt, cnt, x, wl, wr, b.reshape(1, D))


def _final_body(a, cn, ht, wl, wr, b, gnw, gnb, gms, pw1, pb1, pw2, pb2, o):
    cnt = jnp.maximum(cn[0, :, 0:1] + cn[1, :, 0:1], 1.0)
    mean = (a[0] + a[1]) / cnt
    x = (jnp.dot(mean, wl[...], preferred_element_type=jnp.float32)
         + jnp.dot(ht[...], wr[...], preferred_element_type=jnp.float32)
         + b[...])
    mu = jnp.mean(x, axis=0, keepdims=True)
    cen = x - mu * gms[...]
    var = jnp.mean(cen * cen, axis=0, keepdims=True)
    x = gnw[...] * cen / jnp.sqrt(var + 1e-5) + gnb[...]
    x = jnp.maximum(jnp.dot(x, pw1[...], preferred_element_type=jnp.float32)
                    + pb1[...], 0.0)
    x = jnp.dot(x, pw2[...], preferred_element_type=jnp.float32) + pb2[...]
    nrm = jnp.sqrt(jnp.sum(x * x, axis=1, keepdims=True))
    o[...] = x / jnp.maximum(nrm, 1e-12)


def _final_stage(agg2, cntb, h_tab, wl, wr, b,
                 gnw, gnb, gms, pw1, pb1, pw2, pb2):
    r1 = lambda v: v.reshape(1, D)
    return pl.pallas_call(
        _final_body,
        grid=(1,),
        in_specs=[
            pl.BlockSpec((NC, N, D), lambda i: (0, 0, 0)),
            pl.BlockSpec((NC, N, CW), lambda i: (0, 0, 0)),
            pl.BlockSpec((N, D), lambda i: (0, 0)),
            pl.BlockSpec((D, D), lambda i: (0, 0)),
            pl.BlockSpec((D, D), lambda i: (0, 0)),
            pl.BlockSpec((1, D), lambda i: (0, 0)),
            pl.BlockSpec((1, D), lambda i: (0, 0)),
            pl.BlockSpec((1, D), lambda i: (0, 0)),
            pl.BlockSpec((1, D), lambda i: (0, 0)),
            pl.BlockSpec((D, D), lambda i: (0, 0)),
            pl.BlockSpec((1, D), lambda i: (0, 0)),
            pl.BlockSpec((D, D), lambda i: (0, 0)),
            pl.BlockSpec((1, D), lambda i: (0, 0)),
        ],
        out_specs=pl.BlockSpec((N, D), lambda i: (0, 0)),
        out_shape=jax.ShapeDtypeStruct((N, D), jnp.float32),
    )(agg2, cntb, h_tab, wl, wr, r1(b), r1(gnw), r1(gnb), r1(gms),
      pw1, r1(pb1), pw2, r1(pb2))


def kernel(x_table, x_column, edge_index_t2c, edge_index_c2t,
           Wl1_t2c, Wr1_t2c, b1_t2c, Wl1_c2t, Wr1_c2t, b1_c2t,
           Wl2_t2c, Wr2_t2c, b2_t2c, Wl2_c2t, Wr2_c2t, b2_c2t,
           gn_weight, gn_bias, gn_mean_scale, pW1, pb1, pW2, pb2):
    srcA, dstA = _pad_edges(edge_index_t2c)
    srcB, dstB = _pad_edges(edge_index_c2t)

    aggA, cntA = _conv_cnt(x_table, srcA, dstA)
    aggB, cntB = _conv_cnt(x_column, srcB, dstB)

    h_col = _sage_linear(aggA, cntA, x_column, Wl1_t2c, Wr1_t2c, b1_t2c)
    h_tab = _sage_linear(aggB, cntB, x_table, Wl1_c2t, Wr1_c2t, b1_c2t)

    (agg2,) = _conv_nocnt(h_col, srcB, dstB)

    return _final_stage(agg2, cntB, h_tab, Wl2_c2t, Wr2_c2t, b2_c2t,
                        gn_weight, gn_bias, gn_mean_scale,
                        pW1, pb1, pW2, pb2)


# speedup vs baseline: 2.9211x; 2.9211x over previous
import jax
import jax.numpy as jnp
from jax import lax
from jax.experimental import pallas as pl
from jax.experimental.pallas import tpu as pltpu
from jax.experimental.pallas import tpu_sc as plsc

N = 10000
E = 160000
D = 128
NC = 2
NS = 16
NW = NC * NS
C = 128
NCHUNK = 40
EW = C * NCHUNK
EPAD = EW * NW
PAD_DST = N
ACC = 10240
RPT = ACC // NS
CW = 16

_mesh = plsc.VectorSubcoreMesh(
    core_axis_name="c", subcore_axis_name="s", num_cores=NC, num_subcores=NS)


def _zero_rows(rows):
    zeros16 = jnp.zeros((16,), jnp.float32)

    def _z(i, _):
        rows[i // (D // 16), pl.ds((i % (D // 16)) * 16, 16)] = zeros16
        return 0
    lax.fori_loop(0, C * (D // 16), _z, 0)


def _conv_body(x_hbm, src_hbm, dst_hbm, out_hbm,
               idx_s, idx_d, idx_g, idx_c, rows, acc, sem):
    c = lax.axis_index("c")
    s = lax.axis_index("s")
    w = s * NC + c

    _zero_rows(rows)

    def _zacc(i, _):
        pltpu.sync_copy(rows, acc.at[pl.ds(s * RPT + i * C, C)])
        return 0
    lax.fori_loop(0, RPT // C, _zacc, 0)

    plsc.subcore_barrier()

    pltpu.sync_copy(src_hbm.at[w], idx_s)
    pltpu.sync_copy(dst_hbm.at[w], idx_d)

    def _step(j, _):
        def _stage(k, _):
            idx_g[pl.ds(k * 16, 16)] = idx_s[j, pl.ds(k * 16, 16)]
            idx_c[pl.ds(k * 16, 16)] = idx_d[j, pl.ds(k * 16, 16)]
            return 0
        lax.fori_loop(0, C // 16, _stage, 0)
        pltpu.async_copy(x_hbm.at[idx_g], rows, sem).wait()
        pltpu.sync_copy(rows, acc.at[idx_c], add=True)
        return 0
    lax.fori_loop(0, NCHUNK, _step, 0)

    plsc.subcore_barrier()

    def _rd(i, _):
        pltpu.sync_copy(acc.at[pl.ds(s * RPT + i * C, C)], rows)
        pltpu.sync_copy(rows, out_hbm.at[c, pl.ds(s * RPT + i * C, C)])
        return 0
    lax.fori_loop(0, RPT // C, _rd, 0)


_conv = pl.kernel(
    _conv_body,
    out_type=[
        jax.ShapeDtypeStruct((NC, ACC, D), jnp.float32),
    ],
    mesh=_mesh,
    scratch_types=[
        pltpu.VMEM((NCHUNK, C), jnp.int32),
        pltpu.VMEM((NCHUNK, C), jnp.int32),
        pltpu.VMEM((C,), jnp.int32),
        pltpu.VMEM((C,), jnp.int32),
        pltpu.VMEM((C, D), jnp.float32),
        pltpu.VMEM_SHARED((ACC, D), jnp.float32),
        pltpu.SemaphoreType.DMA,
    ],
)


def _pad_edges(ei):
    pad = EPAD - E
    src = jnp.concatenate([ei[0], jnp.zeros((pad,), jnp.int32)])
    dst = jnp.concatenate([ei[1], jnp.full((pad,), PAD_DST, jnp.int32)])
    return src.reshape(NW, NCHUNK, C), dst.reshape(NW, NCHUNK, C)


_EB = 1000


def _cnt_tc_body(da, db, oa, ob):
    i = pl.program_id(0)

    @pl.when(i == 0)
    def _init():
        oa[...] = jnp.zeros_like(oa)
        ob[...] = jnp.zeros_like(ob)

    lo_iota = lax.broadcasted_iota(jnp.int32, (_EB, D), 1)

    def hist(d):
        lo = (d % D == lo_iota).astype(jnp.float32)
        hi = (d // D == lo_iota).astype(jnp.float32)
        return lax.dot_general(hi, lo, (((0,), (0,)), ((), ())),
                               preferred_element_type=jnp.float32)

    oa[...] += hist(da[...])
    ob[...] += hist(db[...])


def _cnt_tc(dsta, dstb):
    return pl.pallas_call(
        _cnt_tc_body,
        grid=(E // _EB,),
        in_specs=[
            pl.BlockSpec((_EB, 1), lambda i: (i, 0)),
            pl.BlockSpec((_EB, 1), lambda i: (i, 0)),
        ],
        out_specs=[
            pl.BlockSpec((D, D), lambda i: (0, 0)),
            pl.BlockSpec((D, D), lambda i: (0, 0)),
        ],
        out_shape=[
            jax.ShapeDtypeStruct((D, D), jnp.float32),
            jax.ShapeDtypeStruct((D, D), jnp.float32),
        ],
    )(dsta.reshape(E, 1), dstb.reshape(E, 1))


_RB = 400


def _lin_body(a0, a1, cn, x, wl, wr, b, o):
    cnt = jnp.maximum(cn[...], 1.0)
    mean = (a0[0] + a1[0]) / cnt
    h = (jnp.dot(mean, wl[...], preferred_element_type=jnp.float32)
         + jnp.dot(x[...], wr[...], preferred_element_type=jnp.float32)
         + b[...])
    o[...] = jnp.maximum(h, 0.0)


def _sage_linear(agg, cnt, x, wl, wr, b):
    return pl.pallas_call(
        _lin_body,
        grid=(N // _RB,),
        in_specs=[
            pl.BlockSpec((1, _RB, D), lambda i: (0, i, 0)),
            pl.BlockSpec((1, _RB, D), lambda i: (1, i, 0)),
            pl.BlockSpec((_RB, 1), lambda i: (i, 0)),
            pl.BlockSpec((_RB, D), lambda i: (i, 0)),
            pl.BlockSpec((D, D), lambda i: (0, 0)),
            pl.BlockSpec((D, D), lambda i: (0, 0)),
            pl.BlockSpec((1, D), lambda i: (0, 0)),
        ],
        out_specs=pl.BlockSpec((_RB, D), lambda i: (i, 0)),
        out_shape=jax.ShapeDtypeStruct((N, D), jnp.float32),
    )(agg, agg, cnt, x, wl, wr, b.reshape(1, D))


def _final_body(a, cn, ht, wl, wr, b, gnw, gnb, gms, pw1, pb1, pw2, pb2, o):
    cnt = jnp.maximum(cn[...], 1.0)
    mean = (a[0] + a[1]) / cnt
    x = (jnp.dot(mean, wl[...], preferred_element_type=jnp.float32)
         + jnp.dot(ht[...], wr[...], preferred_element_type=jnp.float32)
         + b[...])
    mu = jnp.mean(x, axis=0, keepdims=True)
    cen = x - mu * gms[...]
    var = jnp.mean(cen * cen, axis=0, keepdims=True)
    x = gnw[...] * cen / jnp.sqrt(var + 1e-5) + gnb[...]
    x = jnp.maximum(jnp.dot(x, pw1[...], preferred_element_type=jnp.float32)
                    + pb1[...], 0.0)
    x = jnp.dot(x, pw2[...], preferred_element_type=jnp.float32) + pb2[...]
    nrm = jnp.sqrt(jnp.sum(x * x, axis=1, keepdims=True))
    o[...] = x / jnp.maximum(nrm, 1e-12)


def _final_stage(agg2, cntb, h_tab, wl, wr, b,
                 gnw, gnb, gms, pw1, pb1, pw2, pb2):
    r1 = lambda v: v.reshape(1, D)
    return pl.pallas_call(
        _final_body,
        grid=(1,),
        in_specs=[
            pl.BlockSpec((NC, N, D), lambda i: (0, 0, 0)),
            pl.BlockSpec((N, 1), lambda i: (0, 0)),
            pl.BlockSpec((N, D), lambda i: (0, 0)),
            pl.BlockSpec((D, D), lambda i: (0, 0)),
            pl.BlockSpec((D, D), lambda i: (0, 0)),
            pl.BlockSpec((1, D), lambda i: (0, 0)),
            pl.BlockSpec((1, D), lambda i: (0, 0)),
            pl.BlockSpec((1, D), lambda i: (0, 0)),
            pl.BlockSpec((1, D), lambda i: (0, 0)),
            pl.BlockSpec((D, D), lambda i: (0, 0)),
            pl.BlockSpec((1, D), lambda i: (0, 0)),
            pl.BlockSpec((D, D), lambda i: (0, 0)),
            pl.BlockSpec((1, D), lambda i: (0, 0)),
        ],
        out_specs=pl.BlockSpec((N, D), lambda i: (0, 0)),
        out_shape=jax.ShapeDtypeStruct((N, D), jnp.float32),
    )(agg2, cntb, h_tab, wl, wr, r1(b), r1(gnw), r1(gnb), r1(gms),
      pw1, r1(pb1), pw2, r1(pb2))


def kernel(x_table, x_column, edge_index_t2c, edge_index_c2t,
           Wl1_t2c, Wr1_t2c, b1_t2c, Wl1_c2t, Wr1_c2t, b1_c2t,
           Wl2_t2c, Wr2_t2c, b2_t2c, Wl2_c2t, Wr2_c2t, b2_c2t,
           gn_weight, gn_bias, gn_mean_scale, pW1, pb1, pW2, pb2):
    srcA, dstA = _pad_edges(edge_index_t2c)
    srcB, dstB = _pad_edges(edge_index_c2t)

    cntA_m, cntB_m = _cnt_tc(edge_index_t2c[1], edge_index_c2t[1])
    cntA = cntA_m.reshape(D * D, 1)
    cntB = cntB_m.reshape(D * D, 1)
    (aggA,) = _conv(x_table, srcA, dstA)
    (aggB,) = _conv(x_column, srcB, dstB)

    h_col = _sage_linear(aggA, cntA, x_column, Wl1_t2c, Wr1_t2c, b1_t2c)
    h_tab = _sage_linear(aggB, cntB, x_table, Wl1_c2t, Wr1_c2t, b1_c2t)

    (agg2,) = _conv(h_col, srcB, dstB)

    return _final_stage(agg2, cntB, h_tab, Wl2_c2t, Wr2_c2t, b2_c2t,
                        gn_weight, gn_bias, gn_mean_scale,
                        pW1, pb1, pW2, pb2)


# speedup vs baseline: 3.2444x; 1.1107x over previous
import jax
import jax.numpy as jnp
from jax import lax
from jax.experimental import pallas as pl
from jax.experimental.pallas import tpu as pltpu
from jax.experimental.pallas import tpu_sc as plsc

N = 10000
E = 160000
D = 128
NC = 2
NS = 16
NW = NC * NS
C = 128
NCHUNK = 40
EW = C * NCHUNK
EPAD = EW * NW
PAD_DST = N
ACC = 10240
RPT = ACC // NS
CW = 16

_mesh = plsc.VectorSubcoreMesh(
    core_axis_name="c", subcore_axis_name="s", num_cores=NC, num_subcores=NS)


def _zero_rows(rows):
    zeros16 = jnp.zeros((16,), jnp.float32)

    def _z(i, _):
        rows[i // (D // 16), pl.ds((i % (D // 16)) * 16, 16)] = zeros16
        return 0
    lax.fori_loop(0, C * (D // 16), _z, 0)


def _conv_body(x_hbm, src_hbm, dst_hbm, out_hbm,
               idx_s, idx_d, g0, c0, g1, c1, rows0, rows1, acc,
               sem0, sem1):
    cx = lax.axis_index("c")
    s = lax.axis_index("s")
    w = s * NC + cx

    pltpu.sync_copy(src_hbm.at[w], idx_s)
    pltpu.sync_copy(dst_hbm.at[w], idx_d)

    def _stage(j, g, cc):
        def _k(k, _):
            g[pl.ds(k * 16, 16)] = idx_s[j, pl.ds(k * 16, 16)]
            cc[pl.ds(k * 16, 16)] = idx_d[j, pl.ds(k * 16, 16)]
            return 0
        lax.fori_loop(0, C // 16, _k, 0)

    _stage(0, g0, c0)
    pltpu.async_copy(x_hbm.at[g0], rows0, sem0)

    _zero_rows(rows1)

    def _zacc(i, _):
        pltpu.sync_copy(rows1, acc.at[pl.ds(s * RPT + i * C, C)])
        return 0
    lax.fori_loop(0, RPT // C, _zacc, 0)

    plsc.subcore_barrier()

    def _body(jj, _):
        j1 = 2 * jj + 1
        _stage(j1, g1, c1)
        pltpu.async_copy(x_hbm.at[g1], rows1, sem1)
        pltpu.make_async_copy(x_hbm.at[g0], rows0, sem0).wait()
        pltpu.sync_copy(rows0, acc.at[c0], add=True)

        j2 = 2 * jj + 2

        @pl.when(j2 < NCHUNK)
        def _():
            _stage(j2, g0, c0)
            pltpu.async_copy(x_hbm.at[g0], rows0, sem0)

        pltpu.make_async_copy(x_hbm.at[g1], rows1, sem1).wait()
        pltpu.sync_copy(rows1, acc.at[c1], add=True)
        return 0
    lax.fori_loop(0, NCHUNK // 2, _body, 0)

    plsc.subcore_barrier()

    def _rd(i, _):
        pltpu.sync_copy(acc.at[pl.ds(s * RPT + i * C, C)], rows0)
        pltpu.sync_copy(rows0, out_hbm.at[cx, pl.ds(s * RPT + i * C, C)])
        return 0
    lax.fori_loop(0, RPT // C, _rd, 0)


_conv = pl.kernel(
    _conv_body,
    out_type=[
        jax.ShapeDtypeStruct((NC, ACC, D), jnp.float32),
    ],
    mesh=_mesh,
    scratch_types=[
        pltpu.VMEM((NCHUNK, C), jnp.int32),
        pltpu.VMEM((NCHUNK, C), jnp.int32),
        pltpu.VMEM((C,), jnp.int32),
        pltpu.VMEM((C,), jnp.int32),
        pltpu.VMEM((C,), jnp.int32),
        pltpu.VMEM((C,), jnp.int32),
        pltpu.VMEM((C, D), jnp.float32),
        pltpu.VMEM((C, D), jnp.float32),
        pltpu.VMEM_SHARED((ACC, D), jnp.float32),
        pltpu.SemaphoreType.DMA,
        pltpu.SemaphoreType.DMA,
    ],
)


def _pad_edges(ei):
    pad = EPAD - E
    src = jnp.concatenate([ei[0], jnp.zeros((pad,), jnp.int32)])
    dst = jnp.concatenate([ei[1], jnp.full((pad,), PAD_DST, jnp.int32)])
    return src.reshape(NW, NCHUNK, C), dst.reshape(NW, NCHUNK, C)


_EB = 1000


def _cnt_tc_body(da, db, oa, ob):
    i = pl.program_id(0)

    @pl.when(i == 0)
    def _init():
        oa[...] = jnp.zeros_like(oa)
        ob[...] = jnp.zeros_like(ob)

    lo_iota = lax.broadcasted_iota(jnp.int32, (_EB, D), 1)

    def hist(d):
        lo = (d % D == lo_iota).astype(jnp.float32)
        hi = (d // D == lo_iota).astype(jnp.float32)
        return lax.dot_general(hi, lo, (((0,), (0,)), ((), ())),
                               preferred_element_type=jnp.float32)

    oa[...] += hist(da[...])
    ob[...] += hist(db[...])


def _cnt_tc(dsta, dstb):
    return pl.pallas_call(
        _cnt_tc_body,
        grid=(E // _EB,),
        in_specs=[
            pl.BlockSpec((_EB, 1), lambda i: (i, 0)),
            pl.BlockSpec((_EB, 1), lambda i: (i, 0)),
        ],
        out_specs=[
            pl.BlockSpec((D, D), lambda i: (0, 0)),
            pl.BlockSpec((D, D), lambda i: (0, 0)),
        ],
        out_shape=[
            jax.ShapeDtypeStruct((D, D), jnp.float32),
            jax.ShapeDtypeStruct((D, D), jnp.float32),
        ],
    )(dsta.reshape(E, 1), dstb.reshape(E, 1))


_RB = 400


def _lin_body(a0, a1, cn, x, wl, wr, b, o):
    cnt = jnp.maximum(cn[...], 1.0)
    mean = (a0[0] + a1[0]) / cnt
    h = (jnp.dot(mean, wl[...], preferred_element_type=jnp.float32)
         + jnp.dot(x[...], wr[...], preferred_element_type=jnp.float32)
         + b[...])
    o[...] = jnp.maximum(h, 0.0)


def _sage_linear(agg, cnt, x, wl, wr, b):
    return pl.pallas_call(
        _lin_body,
        grid=(N // _RB,),
        in_specs=[
            pl.BlockSpec((1, _RB, D), lambda i: (0, i, 0)),
            pl.BlockSpec((1, _RB, D), lambda i: (1, i, 0)),
            pl.BlockSpec((_RB, 1), lambda i: (i, 0)),
            pl.BlockSpec((_RB, D), lambda i: (i, 0)),
            pl.BlockSpec((D, D), lambda i: (0, 0)),
            pl.BlockSpec((D, D), lambda i: (0, 0)),
            pl.BlockSpec((1, D), lambda i: (0, 0)),
        ],
        out_specs=pl.BlockSpec((_RB, D), lambda i: (i, 0)),
        out_shape=jax.ShapeDtypeStruct((N, D), jnp.float32),
    )(agg, agg, cnt, x, wl, wr, b.reshape(1, D))


def _final_body(a, cn, ht, wl, wr, b, gnw, gnb, gms, pw1, pb1, pw2, pb2, o):
    cnt = jnp.maximum(cn[...], 1.0)
    mean = (a[0] + a[1]) / cnt
    x = (jnp.dot(mean, wl[...], preferred_element_type=jnp.float32)
         + jnp.dot(ht[...], wr[...], preferred_element_type=jnp.float32)
         + b[...])
    mu = jnp.mean(x, axis=0, keepdims=True)
    cen = x - mu * gms[...]
    var = jnp.mean(cen * cen, axis=0, keepdims=True)
    x = gnw[...] * cen / jnp.sqrt(var + 1e-5) + gnb[...]
    x = jnp.maximum(jnp.dot(x, pw1[...], preferred_element_type=jnp.float32)
                    + pb1[...], 0.0)
    x = jnp.dot(x, pw2[...], preferred_element_type=jnp.float32) + pb2[...]
    nrm = jnp.sqrt(jnp.sum(x * x, axis=1, keepdims=True))
    o[...] = x / jnp.maximum(nrm, 1e-12)


def _final_stage(agg2, cntb, h_tab, wl, wr, b,
                 gnw, gnb, gms, pw1, pb1, pw2, pb2):
    r1 = lambda v: v.reshape(1, D)
    return pl.pallas_call(
        _final_body,
        grid=(1,),
        in_specs=[
            pl.BlockSpec((NC, N, D), lambda i: (0, 0, 0)),
            pl.BlockSpec((N, 1), lambda i: (0, 0)),
            pl.BlockSpec((N, D), lambda i: (0, 0)),
            pl.BlockSpec((D, D), lambda i: (0, 0)),
            pl.BlockSpec((D, D), lambda i: (0, 0)),
            pl.BlockSpec((1, D), lambda i: (0, 0)),
            pl.BlockSpec((1, D), lambda i: (0, 0)),
            pl.BlockSpec((1, D), lambda i: (0, 0)),
            pl.BlockSpec((1, D), lambda i: (0, 0)),
            pl.BlockSpec((D, D), lambda i: (0, 0)),
            pl.BlockSpec((1, D), lambda i: (0, 0)),
            pl.BlockSpec((D, D), lambda i: (0, 0)),
            pl.BlockSpec((1, D), lambda i: (0, 0)),
        ],
        out_specs=pl.BlockSpec((N, D), lambda i: (0, 0)),
        out_shape=jax.ShapeDtypeStruct((N, D), jnp.float32),
    )(agg2, cntb, h_tab, wl, wr, r1(b), r1(gnw), r1(gnb), r1(gms),
      pw1, r1(pb1), pw2, r1(pb2))


def kernel(x_table, x_column, edge_index_t2c, edge_index_c2t,
           Wl1_t2c, Wr1_t2c, b1_t2c, Wl1_c2t, Wr1_c2t, b1_c2t,
           Wl2_t2c, Wr2_t2c, b2_t2c, Wl2_c2t, Wr2_c2t, b2_c2t,
           gn_weight, gn_bias, gn_mean_scale, pW1, pb1, pW2, pb2):
    srcA, dstA = _pad_edges(edge_index_t2c)
    srcB, dstB = _pad_edges(edge_index_c2t)

    cntA_m, cntB_m = _cnt_tc(edge_index_t2c[1], edge_index_c2t[1])
    cntA = cntA_m.reshape(D * D, 1)
    cntB = cntB_m.reshape(D * D, 1)
    (aggA,) = _conv(x_table, srcA, dstA)
    (aggB,) = _conv(x_column, srcB, dstB)

    h_col = _sage_linear(aggA, cntA, x_column, Wl1_t2c, Wr1_t2c, b1_t2c)
    h_tab = _sage_linear(aggB, cntB, x_table, Wl1_c2t, Wr1_c2t, b1_c2t)

    (agg2,) = _conv(h_col, srcB, dstB)

    return _final_stage(agg2, cntB, h_tab, Wl2_c2t, Wr2_c2t, b2_c2t,
                        gn_weight, gn_bias, gn_mean_scale,
                        pW1, pb1, pW2, pb2)
